# baseline (speedup 1.0000x reference)
import functools

import jax
import jax.numpy as jnp
from jax import lax
from jax.experimental import pallas as pl
from jax.experimental.pallas import tpu as pltpu

N_TOK = 2048
D_IN = 2048
D_MKT = 16
N_EXP = 8
TOP_K = 2
D_H = 1024
D_OUT = 512

TB = 128
N_TB = N_TOK // TB

_INV_SQRT2 = 0.7071067811865476


def _gate_body(x_ref, mkt_ref, wgx_ref, wgm_ref, bg_ref, probs_ref, wd_ref):
    logits = (jnp.dot(x_ref[...], wgx_ref[...], preferred_element_type=jnp.float32)
              + jnp.dot(mkt_ref[...], wgm_ref[...], preferred_element_type=jnp.float32)
              + bg_ref[...])
    z = logits - jnp.max(logits, axis=-1, keepdims=True)
    p = jnp.exp(z)
    p = p / jnp.sum(p, axis=-1, keepdims=True)
    probs_ref[...] = p

    iota = lax.broadcasted_iota(jnp.int32, p.shape, 1)
    m1 = jnp.max(p, axis=-1, keepdims=True)
    idx1 = jnp.min(jnp.where(p == m1, iota, N_EXP), axis=-1, keepdims=True)
    p2 = jnp.where(iota == idx1, -jnp.inf, p)
    m2 = jnp.max(p2, axis=-1, keepdims=True)
    idx2 = jnp.min(jnp.where(p2 == m2, iota, N_EXP), axis=-1, keepdims=True)
    denom = m1 + m2 + 1e-8
    w1 = m1 / denom
    w2 = m2 / denom
    wd_ref[...] = (jnp.where(iota == idx1, w1, 0.0)
                   + jnp.where(iota == idx2, w2, 0.0))


def _ffn_body(x_ref, w1_ref, b1_ref, w2_ref, b2_ref, wd_ref, out_ref):
    e = pl.program_id(0)
    tb = pl.program_id(1)
    h = jnp.dot(x_ref[...], w1_ref[0], preferred_element_type=jnp.float32) + b1_ref[0]
    h = 0.5 * h * (1.0 + lax.erf(h * _INV_SQRT2))
    y = jnp.dot(h.astype(jnp.bfloat16), w2_ref[0],
                preferred_element_type=jnp.float32) + b2_ref[0]
    iota = lax.broadcasted_iota(jnp.int32, (1, N_EXP), 1)
    w_e = jnp.sum(wd_ref[...] * (iota == e).astype(jnp.float32), axis=-1,
                  keepdims=True)
    contrib = w_e * y
    row = tb * TB

    @pl.when(e == 0)
    def _():
        out_ref[pl.ds(row, TB), :] = contrib

    @pl.when(e != 0)
    def _():
        out_ref[pl.ds(row, TB), :] = out_ref[pl.ds(row, TB), :] + contrib


@jax.jit
def kernel(x, market_status, W_g, b_g, W1, b1, W2, b2):
    wgx = W_g[:D_IN]
    wgm = W_g[D_IN:]
    bg2 = b_g.reshape(1, N_EXP)

    probs, wd = pl.pallas_call(
        _gate_body,
        grid=(N_TB,),
        in_specs=[
            pl.BlockSpec((TB, D_IN), lambda t: (t, 0)),
            pl.BlockSpec((TB, D_MKT), lambda t: (t, 0)),
            pl.BlockSpec((D_IN, N_EXP), lambda t: (0, 0)),
            pl.BlockSpec((D_MKT, N_EXP), lambda t: (0, 0)),
            pl.BlockSpec((1, N_EXP), lambda t: (0, 0)),
        ],
        out_specs=[
            pl.BlockSpec((TB, N_EXP), lambda t: (t, 0)),
            pl.BlockSpec((TB, N_EXP), lambda t: (t, 0)),
        ],
        out_shape=[
            jax.ShapeDtypeStruct((N_TOK, N_EXP), jnp.float32),
            jax.ShapeDtypeStruct((N_TOK, N_EXP), jnp.float32),
        ],
    )(x, market_status, wgx, wgm, bg2)

    x_bf = x.astype(jnp.bfloat16)
    W1_bf = W1.astype(jnp.bfloat16)
    W2_bf = W2.astype(jnp.bfloat16)

    out = pl.pallas_call(
        _ffn_body,
        grid=(N_EXP, N_TB),
        in_specs=[
            pl.BlockSpec((TB, D_IN), lambda e, t: (t, 0)),
            pl.BlockSpec((1, D_IN, D_H), lambda e, t: (e, 0, 0)),
            pl.BlockSpec((1, 1, D_H), lambda e, t: (e, 0, 0)),
            pl.BlockSpec((1, D_H, D_OUT), lambda e, t: (e, 0, 0)),
            pl.BlockSpec((1, 1, D_OUT), lambda e, t: (e, 0, 0)),
            pl.BlockSpec((TB, N_EXP), lambda e, t: (t, 0)),
        ],
        out_specs=pl.BlockSpec((N_TOK, D_OUT), lambda e, t: (0, 0)),
        out_shape=jax.ShapeDtypeStruct((N_TOK, D_OUT), jnp.float32),
    )(x_bf, W1_bf, b1.reshape(N_EXP, 1, D_H), W2_bf,
      b2.reshape(N_EXP, 1, D_OUT), wd)

    return out, probs


# speedup vs baseline: 1.4608x; 1.4608x over previous
import functools

import jax
import jax.numpy as jnp
from jax import lax
from jax.experimental import pallas as pl
from jax.experimental.pallas import tpu as pltpu

N_TOK = 2048
D_IN = 2048
D_MKT = 16
N_EXP = 8
TOP_K = 2
D_H = 1024
D_OUT = 512

TB = 256
N_TB = N_TOK // TB

_INV_SQRT2 = 0.7071067811865476


def _gate_body(x_ref, mkt_ref, wgx_ref, wgm_ref, bg_ref, probs_ref, wd_ref):
    logits = (jnp.dot(x_ref[...], wgx_ref[...], preferred_element_type=jnp.float32)
              + jnp.dot(mkt_ref[...], wgm_ref[...], preferred_element_type=jnp.float32)
              + bg_ref[...])
    z = logits - jnp.max(logits, axis=-1, keepdims=True)
    p = jnp.exp(z)
    p = p / jnp.sum(p, axis=-1, keepdims=True)
    probs_ref[...] = p

    iota = lax.broadcasted_iota(jnp.int32, p.shape, 1)
    m1 = jnp.max(p, axis=-1, keepdims=True)
    idx1 = jnp.min(jnp.where(p == m1, iota, N_EXP), axis=-1, keepdims=True)
    p2 = jnp.where(iota == idx1, -jnp.inf, p)
    m2 = jnp.max(p2, axis=-1, keepdims=True)
    idx2 = jnp.min(jnp.where(p2 == m2, iota, N_EXP), axis=-1, keepdims=True)
    denom = m1 + m2 + 1e-8
    w1 = m1 / denom
    w2 = m2 / denom
    wd_ref[...] = (jnp.where(iota == idx1, w1, 0.0)
                   + jnp.where(iota == idx2, w2, 0.0))


def _ffn_body(x_ref, w1_ref, b1_ref, w2_ref, b2_ref, wd_ref, out_ref):
    e = pl.program_id(0)
    tb = pl.program_id(1)
    h = jnp.dot(x_ref[...], w1_ref[0], preferred_element_type=jnp.float32) + b1_ref[0]
    h = 0.5 * h * (1.0 + lax.erf(h * _INV_SQRT2))
    y = jnp.dot(h, w2_ref[0], preferred_element_type=jnp.float32) + b2_ref[0]
    iota = lax.broadcasted_iota(jnp.int32, (1, N_EXP), 1)
    w_e = jnp.sum(wd_ref[...] * (iota == e).astype(jnp.float32), axis=-1,
                  keepdims=True)
    contrib = w_e * y
    row = tb * TB

    @pl.when(e == 0)
    def _():
        out_ref[pl.ds(row, TB), :] = contrib

    @pl.when(e != 0)
    def _():
        out_ref[pl.ds(row, TB), :] = out_ref[pl.ds(row, TB), :] + contrib


@jax.jit
def kernel(x, market_status, W_g, b_g, W1, b1, W2, b2):
    wgx = W_g[:D_IN]
    wgm = W_g[D_IN:]
    bg2 = b_g.reshape(1, N_EXP)

    probs, wd = pl.pallas_call(
        _gate_body,
        grid=(N_TB,),
        in_specs=[
            pl.BlockSpec((TB, D_IN), lambda t: (t, 0)),
            pl.BlockSpec((TB, D_MKT), lambda t: (t, 0)),
            pl.BlockSpec((D_IN, N_EXP), lambda t: (0, 0)),
            pl.BlockSpec((D_MKT, N_EXP), lambda t: (0, 0)),
            pl.BlockSpec((1, N_EXP), lambda t: (0, 0)),
        ],
        out_specs=[
            pl.BlockSpec((TB, N_EXP), lambda t: (t, 0)),
            pl.BlockSpec((TB, N_EXP), lambda t: (t, 0)),
        ],
        out_shape=[
            jax.ShapeDtypeStruct((N_TOK, N_EXP), jnp.float32),
            jax.ShapeDtypeStruct((N_TOK, N_EXP), jnp.float32),
        ],
    )(x, market_status, wgx, wgm, bg2)

    out = pl.pallas_call(
        _ffn_body,
        grid=(N_EXP, N_TB),
        in_specs=[
            pl.BlockSpec((TB, D_IN), lambda e, t: (t, 0)),
            pl.BlockSpec((1, D_IN, D_H), lambda e, t: (e, 0, 0)),
            pl.BlockSpec((1, 1, D_H), lambda e, t: (e, 0, 0)),
            pl.BlockSpec((1, D_H, D_OUT), lambda e, t: (e, 0, 0)),
            pl.BlockSpec((1, 1, D_OUT), lambda e, t: (e, 0, 0)),
            pl.BlockSpec((TB, N_EXP), lambda e, t: (t, 0)),
        ],
        out_specs=pl.BlockSpec((N_TOK, D_OUT), lambda e, t: (0, 0)),
        out_shape=jax.ShapeDtypeStruct((N_TOK, D_OUT), jnp.float32),
    )(x, W1, b1.reshape(N_EXP, 1, D_H), W2,
      b2.reshape(N_EXP, 1, D_OUT), wd)

    return out, probs


# speedup vs baseline: 1.6783x; 1.1489x over previous
import functools

import jax
import jax.numpy as jnp
from jax import lax
from jax.experimental import pallas as pl
from jax.experimental.pallas import tpu as pltpu

N_TOK = 2048
D_IN = 2048
D_MKT = 16
N_EXP = 8
TOP_K = 2
D_H = 1024
D_OUT = 512

TB = 512
N_TB = N_TOK // TB

_INV_SQRT2 = 0.7071067811865476


def _gate_body(x_ref, mkt_ref, wgx_ref, wgm_ref, bg_ref, probs_ref, wd_ref):
    logits = (jnp.dot(x_ref[...], wgx_ref[...], preferred_element_type=jnp.float32)
              + jnp.dot(mkt_ref[...], wgm_ref[...], preferred_element_type=jnp.float32)
              + bg_ref[...])
    z = logits - jnp.max(logits, axis=-1, keepdims=True)
    p = jnp.exp(z)
    p = p / jnp.sum(p, axis=-1, keepdims=True)
    probs_ref[...] = p

    iota = lax.broadcasted_iota(jnp.int32, p.shape, 1)
    m1 = jnp.max(p, axis=-1, keepdims=True)
    idx1 = jnp.min(jnp.where(p == m1, iota, N_EXP), axis=-1, keepdims=True)
    p2 = jnp.where(iota == idx1, -jnp.inf, p)
    m2 = jnp.max(p2, axis=-1, keepdims=True)
    idx2 = jnp.min(jnp.where(p2 == m2, iota, N_EXP), axis=-1, keepdims=True)
    denom = m1 + m2 + 1e-8
    w1 = m1 / denom
    w2 = m2 / denom
    wd_ref[...] = (jnp.where(iota == idx1, w1, 0.0)
                   + jnp.where(iota == idx2, w2, 0.0))


def _ffn_body(x_ref, w1_ref, b1_ref, w2_ref, b2_ref, wd_ref, out_ref):
    e = pl.program_id(0)
    tb = pl.program_id(1)
    h = jnp.dot(x_ref[...], w1_ref[0], preferred_element_type=jnp.float32) + b1_ref[0]
    h = 0.5 * h * (1.0 + lax.erf(h * _INV_SQRT2))
    y = jnp.dot(h, w2_ref[0], preferred_element_type=jnp.float32) + b2_ref[0]
    iota = lax.broadcasted_iota(jnp.int32, (1, N_EXP), 1)
    w_e = jnp.sum(wd_ref[...] * (iota == e).astype(jnp.float32), axis=-1,
                  keepdims=True)
    contrib = w_e * y
    row = tb * TB

    @pl.when(e == 0)
    def _():
        out_ref[pl.ds(row, TB), :] = contrib

    @pl.when(e != 0)
    def _():
        out_ref[pl.ds(row, TB), :] = out_ref[pl.ds(row, TB), :] + contrib


@jax.jit
def kernel(x, market_status, W_g, b_g, W1, b1, W2, b2):
    wgx = W_g[:D_IN]
    wgm = W_g[D_IN:]
    bg2 = b_g.reshape(1, N_EXP)

    probs, wd = pl.pallas_call(
        _gate_body,
        grid=(N_TB,),
        in_specs=[
            pl.BlockSpec((TB, D_IN), lambda t: (t, 0)),
            pl.BlockSpec((TB, D_MKT), lambda t: (t, 0)),
            pl.BlockSpec((D_IN, N_EXP), lambda t: (0, 0)),
            pl.BlockSpec((D_MKT, N_EXP), lambda t: (0, 0)),
            pl.BlockSpec((1, N_EXP), lambda t: (0, 0)),
        ],
        out_specs=[
            pl.BlockSpec((TB, N_EXP), lambda t: (t, 0)),
            pl.BlockSpec((TB, N_EXP), lambda t: (t, 0)),
        ],
        out_shape=[
            jax.ShapeDtypeStruct((N_TOK, N_EXP), jnp.float32),
            jax.ShapeDtypeStruct((N_TOK, N_EXP), jnp.float32),
        ],
    )(x, market_status, wgx, wgm, bg2)

    out = pl.pallas_call(
        _ffn_body,
        grid=(N_EXP, N_TB),
        in_specs=[
            pl.BlockSpec((TB, D_IN), lambda e, t: (t, 0)),
            pl.BlockSpec((1, D_IN, D_H), lambda e, t: (e, 0, 0)),
            pl.BlockSpec((1, 1, D_H), lambda e, t: (e, 0, 0)),
            pl.BlockSpec((1, D_H, D_OUT), lambda e, t: (e, 0, 0)),
            pl.BlockSpec((1, 1, D_OUT), lambda e, t: (e, 0, 0)),
            pl.BlockSpec((TB, N_EXP), lambda e, t: (t, 0)),
        ],
        out_specs=pl.BlockSpec((N_TOK, D_OUT), lambda e, t: (0, 0)),
        out_shape=jax.ShapeDtypeStruct((N_TOK, D_OUT), jnp.float32),
    )(x, W1, b1.reshape(N_EXP, 1, D_H), W2,
      b2.reshape(N_EXP, 1, D_OUT), wd)

    return out, probs


# speedup vs baseline: 1.8769x; 1.1183x over previous
import functools

import jax
import jax.numpy as jnp
from jax import lax
from jax.experimental import pallas as pl
from jax.experimental.pallas import tpu as pltpu

N_TOK = 2048
D_IN = 2048
D_MKT = 16
N_EXP = 8
TOP_K = 2
D_H = 1024
D_OUT = 512

TB = 1024
N_TB = N_TOK // TB

_INV_SQRT2 = 0.7071067811865476


def _gate_body(x_ref, mkt_ref, wgx_ref, wgm_ref, bg_ref, probs_ref, wd_ref):
    logits = (jnp.dot(x_ref[...], wgx_ref[...], preferred_element_type=jnp.float32)
              + jnp.dot(mkt_ref[...], wgm_ref[...], preferred_element_type=jnp.float32)
              + bg_ref[...])
    z = logits - jnp.max(logits, axis=-1, keepdims=True)
    p = jnp.exp(z)
    p = p / jnp.sum(p, axis=-1, keepdims=True)
    probs_ref[...] = p

    iota = lax.broadcasted_iota(jnp.int32, p.shape, 1)
    m1 = jnp.max(p, axis=-1, keepdims=True)
    idx1 = jnp.min(jnp.where(p == m1, iota, N_EXP), axis=-1, keepdims=True)
    p2 = jnp.where(iota == idx1, -jnp.inf, p)
    m2 = jnp.max(p2, axis=-1, keepdims=True)
    idx2 = jnp.min(jnp.where(p2 == m2, iota, N_EXP), axis=-1, keepdims=True)
    denom = m1 + m2 + 1e-8
    w1 = m1 / denom
    w2 = m2 / denom
    wd_ref[...] = (jnp.where(iota == idx1, w1, 0.0)
                   + jnp.where(iota == idx2, w2, 0.0))


def _ffn_body(x_ref, w1_ref, b1_ref, w2_ref, b2_ref, wd_ref, out_ref):
    e = pl.program_id(0)
    tb = pl.program_id(1)
    h = jnp.dot(x_ref[...], w1_ref[0], preferred_element_type=jnp.float32) + b1_ref[0]
    h = 0.5 * h * (1.0 + lax.erf(h * _INV_SQRT2))
    y = jnp.dot(h, w2_ref[0], preferred_element_type=jnp.float32) + b2_ref[0]
    iota = lax.broadcasted_iota(jnp.int32, (1, N_EXP), 1)
    w_e = jnp.sum(wd_ref[...] * (iota == e).astype(jnp.float32), axis=-1,
                  keepdims=True)
    contrib = w_e * y
    row = tb * TB

    @pl.when(e == 0)
    def _():
        out_ref[pl.ds(row, TB), :] = contrib

    @pl.when(e != 0)
    def _():
        out_ref[pl.ds(row, TB), :] = out_ref[pl.ds(row, TB), :] + contrib


@jax.jit
def kernel(x, market_status, W_g, b_g, W1, b1, W2, b2):
    wgx = W_g[:D_IN]
    wgm = W_g[D_IN:]
    bg2 = b_g.reshape(1, N_EXP)

    probs, wd = pl.pallas_call(
        _gate_body,
        grid=(N_TB,),
        in_specs=[
            pl.BlockSpec((TB, D_IN), lambda t: (t, 0)),
            pl.BlockSpec((TB, D_MKT), lambda t: (t, 0)),
            pl.BlockSpec((D_IN, N_EXP), lambda t: (0, 0)),
            pl.BlockSpec((D_MKT, N_EXP), lambda t: (0, 0)),
            pl.BlockSpec((1, N_EXP), lambda t: (0, 0)),
        ],
        out_specs=[
            pl.BlockSpec((TB, N_EXP), lambda t: (t, 0)),
            pl.BlockSpec((TB, N_EXP), lambda t: (t, 0)),
        ],
        out_shape=[
            jax.ShapeDtypeStruct((N_TOK, N_EXP), jnp.float32),
            jax.ShapeDtypeStruct((N_TOK, N_EXP), jnp.float32),
        ],
    )(x, market_status, wgx, wgm, bg2)

    out = pl.pallas_call(
        _ffn_body,
        grid=(N_EXP, N_TB),
        in_specs=[
            pl.BlockSpec((TB, D_IN), lambda e, t: (t, 0)),
            pl.BlockSpec((1, D_IN, D_H), lambda e, t: (e, 0, 0)),
            pl.BlockSpec((1, 1, D_H), lambda e, t: (e, 0, 0)),
            pl.BlockSpec((1, D_H, D_OUT), lambda e, t: (e, 0, 0)),
            pl.BlockSpec((1, 1, D_OUT), lambda e, t: (e, 0, 0)),
            pl.BlockSpec((TB, N_EXP), lambda e, t: (t, 0)),
        ],
        out_specs=pl.BlockSpec((N_TOK, D_OUT), lambda e, t: (0, 0)),
        out_shape=jax.ShapeDtypeStruct((N_TOK, D_OUT), jnp.float32),
    )(x, W1, b1.reshape(N_EXP, 1, D_H), W2,
      b2.reshape(N_EXP, 1, D_OUT), wd)

    return out, probs
